# Initial kernel scaffold; baseline (speedup 1.0000x reference)
import jax
import jax.numpy as jnp
from jax.experimental import pallas as pl


def kernel(inp, W1, b1, gamma, beta, running_mean, running_var, W2, b2):
    raise NotImplementedError("write your pallas kernel here")


# speedup vs baseline: 1.2761x; 1.2761x over previous
import functools

import jax
import jax.numpy as jnp
from jax.experimental import pallas as pl
from jax.experimental.pallas import tpu as pltpu

B, N, H = 512, 32, 100
HP = 128
G = 8


def _body(x_ref, mf_ref, me_ref, b1_ref, w2_ref, cst_ref, out_ref):
    x = x_ref[...]
    xf = x.reshape(G * N, 6)
    f = jnp.dot(xf, mf_ref[...], preferred_element_type=jnp.float32) + b1_ref[...]
    e = jnp.dot(xf, me_ref[...], preferred_element_type=jnp.float32)
    f3 = f.reshape(G, N, HP)
    e3 = e.reshape(G, N, HP)
    sum_e = jnp.sum(e3, axis=1, keepdims=True)
    t = jnp.zeros((G, N, HP), dtype=jnp.float32)
    for j in range(N):
        t = t + jnp.abs(f3 + e3[:, j:j + 1, :])
    diag = f3 + e3
    s = 0.55 * (N * f3 + sum_e) + 0.45 * t - (0.55 * diag + 0.45 * jnp.abs(diag))
    p = jnp.dot(s.reshape(G * N, HP), w2_ref[...],
                preferred_element_type=jnp.float32) + cst_ref[...]
    sp = 0.1 * (jnp.maximum(p, 0.0) + jnp.log1p(jnp.exp(-jnp.abs(p))))
    upd = xf + 0.1 * p
    chan = jax.lax.broadcasted_iota(jnp.int32, (G * N, 6), 1)
    out_ref[...] = jnp.where(chan < 4, upd, sp).reshape(G, N, 6)


@jax.jit
def kernel(inp, W1, b1, gamma, beta, running_mean, running_var, W2, b2):
    f32 = jnp.float32
    inp = inp.astype(f32)
    s = gamma * jax.lax.rsqrt(running_var + 1e-5)
    t = beta - s * running_mean
    w2p = (W2 * s[None, :]).astype(f32)
    cst = (N - 1.0) * (W2 @ t + b2)
    wdy = W1[:, 8]
    wdx = W1[:, 9]
    mf = jnp.concatenate([wdy[:, None], wdx[:, None], W1[:, 0:4]], axis=1)
    me = jnp.concatenate([-wdy[:, None], -wdx[:, None], W1[:, 4:8]], axis=1)
    mf_p = jnp.zeros((6, HP), f32).at[:, :H].set(mf.T)
    me_p = jnp.zeros((6, HP), f32).at[:, :H].set(me.T)
    b1_p = jnp.zeros((1, HP), f32).at[:, :H].set(b1)
    w2_p = jnp.zeros((HP, 6), f32).at[:H, :].set(w2p.T)
    cst_p = cst.reshape(1, 6).astype(f32)

    out = pl.pallas_call(
        _body,
        grid=(B // G,),
        in_specs=[
            pl.BlockSpec((G, N, 6), lambda g: (g, 0, 0)),
            pl.BlockSpec((6, HP), lambda g: (0, 0)),
            pl.BlockSpec((6, HP), lambda g: (0, 0)),
            pl.BlockSpec((1, HP), lambda g: (0, 0)),
            pl.BlockSpec((HP, 6), lambda g: (0, 0)),
            pl.BlockSpec((1, 6), lambda g: (0, 0)),
        ],
        out_specs=pl.BlockSpec((G, N, 6), lambda g: (g, 0, 0)),
        out_shape=jax.ShapeDtypeStruct((B, N, 6), f32),
        compiler_params=pltpu.CompilerParams(
            dimension_semantics=("parallel",)),
    )(inp, mf_p, me_p, b1_p, w2_p, cst_p)
    return out


# speedup vs baseline: 1.5552x; 1.2188x over previous
import functools

import jax
import jax.numpy as jnp
from jax.experimental import pallas as pl
from jax.experimental.pallas import tpu as pltpu

B, N, H = 512, 32, 100
HP = 128
G = 32


def _body(x_ref, mf_ref, me_ref, b1_ref, w2_ref, cst_ref, out_ref):
    x = x_ref[...]
    xf = x.reshape(G * N, 6)
    f = jnp.dot(xf, mf_ref[...], preferred_element_type=jnp.float32) + b1_ref[...]
    e = jnp.dot(xf, me_ref[...], preferred_element_type=jnp.float32)
    s_rows = []
    for g in range(G):
        fg = f[g * N:(g + 1) * N, :]
        eg = e[g * N:(g + 1) * N, :]
        sum_eg = jnp.sum(eg, axis=0, keepdims=True)
        t0 = jnp.abs(fg + eg[0:1, :])
        t1 = jnp.abs(fg + eg[1:2, :])
        for j in range(2, N, 2):
            t0 = t0 + jnp.abs(fg + eg[j:j + 1, :])
            t1 = t1 + jnp.abs(fg + eg[j + 1:j + 2, :])
        diag = fg + eg
        sg = (0.55 * (N * fg + sum_eg) + 0.45 * (t0 + t1)
              - (0.55 * diag + 0.45 * jnp.abs(diag)))
        s_rows.append(sg)
    s = jnp.concatenate(s_rows, axis=0)
    p = jnp.dot(s, w2_ref[...],
                preferred_element_type=jnp.float32) + cst_ref[...]
    sp = 0.1 * (jnp.maximum(p, 0.0) + jnp.log1p(jnp.exp(-jnp.abs(p))))
    upd = xf + 0.1 * p
    chan = jax.lax.broadcasted_iota(jnp.int32, (G * N, 6), 1)
    out_ref[...] = jnp.where(chan < 4, upd, sp).reshape(G, N, 6)


@jax.jit
def kernel(inp, W1, b1, gamma, beta, running_mean, running_var, W2, b2):
    f32 = jnp.float32
    inp = inp.astype(f32)
    s = gamma * jax.lax.rsqrt(running_var + 1e-5)
    t = beta - s * running_mean
    w2p = (W2 * s[None, :]).astype(f32)
    cst = (N - 1.0) * (W2 @ t + b2)
    wdy = W1[:, 8]
    wdx = W1[:, 9]
    mf = jnp.concatenate([wdy[:, None], wdx[:, None], W1[:, 0:4]], axis=1)
    me = jnp.concatenate([-wdy[:, None], -wdx[:, None], W1[:, 4:8]], axis=1)
    mf_p = jnp.zeros((6, HP), f32).at[:, :H].set(mf.T)
    me_p = jnp.zeros((6, HP), f32).at[:, :H].set(me.T)
    b1_p = jnp.zeros((1, HP), f32).at[:, :H].set(b1)
    w2_p = jnp.zeros((HP, 6), f32).at[:H, :].set(w2p.T)
    cst_p = cst.reshape(1, 6).astype(f32)

    out = pl.pallas_call(
        _body,
        grid=(B // G,),
        in_specs=[
            pl.BlockSpec((G, N, 6), lambda g: (g, 0, 0)),
            pl.BlockSpec((6, HP), lambda g: (0, 0)),
            pl.BlockSpec((6, HP), lambda g: (0, 0)),
            pl.BlockSpec((1, HP), lambda g: (0, 0)),
            pl.BlockSpec((HP, 6), lambda g: (0, 0)),
            pl.BlockSpec((1, 6), lambda g: (0, 0)),
        ],
        out_specs=pl.BlockSpec((G, N, 6), lambda g: (g, 0, 0)),
        out_shape=jax.ShapeDtypeStruct((B, N, 6), f32),
        compiler_params=pltpu.CompilerParams(
            dimension_semantics=("parallel",)),
    )(inp, mf_p, me_p, b1_p, w2_p, cst_p)
    return out


# speedup vs baseline: 1.5761x; 1.0134x over previous
import functools

import jax
import jax.numpy as jnp
from jax.experimental import pallas as pl
from jax.experimental.pallas import tpu as pltpu

B, N, H = 512, 32, 100
HP = 128
G = 64


def _body(x_ref, mf_ref, me_ref, b1_ref, w2_ref, cst_ref, out_ref):
    x = x_ref[...]
    xf = x.reshape(G * N, 6)
    f = jnp.dot(xf, mf_ref[...], preferred_element_type=jnp.float32) + b1_ref[...]
    e = jnp.dot(xf, me_ref[...], preferred_element_type=jnp.float32)
    s_rows = []
    for g in range(G):
        fg = f[g * N:(g + 1) * N, :]
        eg = e[g * N:(g + 1) * N, :]
        sum_eg = jnp.sum(eg, axis=0, keepdims=True)
        t0 = jnp.abs(fg + eg[0:1, :])
        t1 = jnp.abs(fg + eg[1:2, :])
        for j in range(2, N, 2):
            t0 = t0 + jnp.abs(fg + eg[j:j + 1, :])
            t1 = t1 + jnp.abs(fg + eg[j + 1:j + 2, :])
        diag = fg + eg
        sg = (0.55 * (N * fg + sum_eg) + 0.45 * (t0 + t1)
              - (0.55 * diag + 0.45 * jnp.abs(diag)))
        s_rows.append(sg)
    s = jnp.concatenate(s_rows, axis=0)
    p = jnp.dot(s, w2_ref[...],
                preferred_element_type=jnp.float32) + cst_ref[...]
    sp = 0.1 * (jnp.maximum(p, 0.0) + jnp.log1p(jnp.exp(-jnp.abs(p))))
    upd = xf + 0.1 * p
    chan = jax.lax.broadcasted_iota(jnp.int32, (G * N, 6), 1)
    out_ref[...] = jnp.where(chan < 4, upd, sp).reshape(G, N, 6)


@jax.jit
def kernel(inp, W1, b1, gamma, beta, running_mean, running_var, W2, b2):
    f32 = jnp.float32
    inp = inp.astype(f32)
    s = gamma * jax.lax.rsqrt(running_var + 1e-5)
    t = beta - s * running_mean
    w2p = (W2 * s[None, :]).astype(f32)
    cst = (N - 1.0) * (W2 @ t + b2)
    wdy = W1[:, 8]
    wdx = W1[:, 9]
    mf = jnp.concatenate([wdy[:, None], wdx[:, None], W1[:, 0:4]], axis=1)
    me = jnp.concatenate([-wdy[:, None], -wdx[:, None], W1[:, 4:8]], axis=1)
    mf_p = jnp.zeros((6, HP), f32).at[:, :H].set(mf.T)
    me_p = jnp.zeros((6, HP), f32).at[:, :H].set(me.T)
    b1_p = jnp.zeros((1, HP), f32).at[:, :H].set(b1)
    w2_p = jnp.zeros((HP, 6), f32).at[:H, :].set(w2p.T)
    cst_p = cst.reshape(1, 6).astype(f32)

    out = pl.pallas_call(
        _body,
        grid=(B // G,),
        in_specs=[
            pl.BlockSpec((G, N, 6), lambda g: (g, 0, 0)),
            pl.BlockSpec((6, HP), lambda g: (0, 0)),
            pl.BlockSpec((6, HP), lambda g: (0, 0)),
            pl.BlockSpec((1, HP), lambda g: (0, 0)),
            pl.BlockSpec((HP, 6), lambda g: (0, 0)),
            pl.BlockSpec((1, 6), lambda g: (0, 0)),
        ],
        out_specs=pl.BlockSpec((G, N, 6), lambda g: (g, 0, 0)),
        out_shape=jax.ShapeDtypeStruct((B, N, 6), f32),
        compiler_params=pltpu.CompilerParams(
            dimension_semantics=("parallel",)),
    )(inp, mf_p, me_p, b1_p, w2_p, cst_p)
    return out
